# baseline (speedup 1.0000x reference)
import functools

import jax
import jax.numpy as jnp
from jax import lax
from jax.experimental import pallas as pl
from jax.experimental.pallas import tpu as pltpu
from jax.experimental.pallas import tpu_sc as plsc

N = 10000
E = 320000
NP = 10240
EP = 327680
NC = 2
NS = 16
GRP = 128
SLAB = 16
SL = (EP // (NC * NS)) // (SLAB * GRP)
RPT = NP // NS
ZR = 64
BN = 512


def _mesh():
    return plsc.VectorSubcoreMesh(core_axis_name="c", subcore_axis_name="s")


NG = EP // (NC * NS) // GRP
NBUF = 2
FS = 64


@functools.lru_cache(None)
def _scatter_kernel(F=FS):
    @functools.partial(
        pl.kernel,
        mesh=_mesh(),
        compiler_params=pltpu.CompilerParams(use_tc_tiling_on_sc=False),
        out_type=jax.ShapeDtypeStruct((NC, NP, F), jnp.float32),
        scratch_types=[
            pltpu.VMEM((NG + NBUF, GRP), jnp.int32),
            pltpu.VMEM((NG, GRP), jnp.int32),
        ] + [pltpu.VMEM((GRP, F), jnp.float32)] * NBUF + [
            pltpu.VMEM_SHARED((NP, F), jnp.float32),
            pltpu.VMEM_SHARED((NP, F), jnp.float32),
        ] + [pltpu.SemaphoreType.DMA] * (2 * NBUF + 2),
    )
    def k(v_hbm, src_hbm, dst_hbm, z_hbm, out_hbm, src_all, dst_all,
          r0, r1, acc, vtab, g0, g1, s0, s1, zsem, isem):
        rows = (r0, r1)
        gsem = (g0, g1)
        ssem = (s0, s1)
        cid = lax.axis_index("c")
        sid = lax.axis_index("s")
        wid = sid * NC + cid
        row0 = wid * NG
        pltpu.make_async_copy(src_hbm.at[pl.ds(row0, NG + NBUF)], src_all,
                              isem).start()
        pltpu.make_async_copy(dst_hbm.at[pl.ds(row0, NG)], dst_all,
                              isem).start()
        pltpu.make_async_copy(v_hbm.at[pl.ds(sid * RPT, RPT)],
                              vtab.at[pl.ds(sid * RPT, RPT)], isem).start()
        pltpu.sync_copy(z_hbm, r0.at[pl.ds(0, ZR)])
        for i in range(RPT // ZR):
            pltpu.make_async_copy(
                r0.at[pl.ds(0, ZR)],
                acc.at[pl.ds(sid * RPT + i * ZR, ZR)], zsem).start()
        for i in range(RPT // ZR):
            pltpu.make_async_copy(
                r0.at[pl.ds(0, ZR)],
                acc.at[pl.ds(sid * RPT + i * ZR, ZR)], zsem).wait()
        pltpu.make_async_copy(src_hbm.at[pl.ds(row0, NG + NBUF)], src_all,
                              isem).wait()
        pltpu.make_async_copy(dst_hbm.at[pl.ds(row0, NG)], dst_all,
                              isem).wait()
        pltpu.make_async_copy(v_hbm.at[pl.ds(sid * RPT, RPT)],
                              vtab.at[pl.ds(sid * RPT, RPT)], isem).wait()
        plsc.subcore_barrier()

        for b in range(NBUF):
            pltpu.make_async_copy(vtab.at[src_all.at[b]], rows[b],
                                  gsem[b]).start()

        def outer(g, carry):
            for b in range(NBUF):
                i = g * NBUF + b
                pltpu.make_async_copy(vtab.at[src_all.at[i]], rows[b],
                                      gsem[b]).wait()
                sc = pltpu.make_async_copy(rows[b], acc.at[dst_all.at[i]],
                                           ssem[b])
                sc.start(add=True)
                sc.wait()
                pltpu.make_async_copy(vtab.at[src_all.at[i + NBUF]], rows[b],
                                      gsem[b]).start()
            return carry

        lax.fori_loop(0, NG // NBUF - 1, outer, 0)
        for b in range(NBUF):
            i = NG - NBUF + b
            pltpu.make_async_copy(vtab.at[src_all.at[i]], rows[b],
                                  gsem[b]).wait()
            sc = pltpu.make_async_copy(rows[b], acc.at[dst_all.at[i]],
                                       ssem[b])
            sc.start(add=True)
            sc.wait()
        plsc.subcore_barrier()
        pltpu.sync_copy(acc.at[pl.ds(sid * RPT, RPT)],
                        out_hbm.at[cid, pl.ds(sid * RPT, RPT)])

    return k


@functools.lru_cache(None)
def _deg_kernel():
    @functools.partial(
        pl.kernel,
        mesh=_mesh(),
        compiler_params=pltpu.CompilerParams(use_tc_tiling_on_sc=False),
        out_type=jax.ShapeDtypeStruct((NC, NP, 16), jnp.float32),
        scratch_types=[
            pltpu.VMEM((NG, GRP), jnp.int32),
            pltpu.VMEM((GRP, 16), jnp.float32),
            pltpu.VMEM((ZR, 16), jnp.float32),
            pltpu.VMEM_SHARED((NP, 16), jnp.float32),
            pltpu.SemaphoreType.DMA,
            pltpu.SemaphoreType.DMA,
        ],
    )
    def k(src_hbm, ones_hbm, z_hbm, out_hbm, src_all, ones_v, zb, acc,
          zsem, ssem):
        cid = lax.axis_index("c")
        sid = lax.axis_index("s")
        wid = sid * NC + cid
        row0 = wid * NG
        pltpu.make_async_copy(src_hbm.at[pl.ds(row0, NG)], src_all,
                              zsem).start()
        pltpu.sync_copy(ones_hbm, ones_v)
        pltpu.sync_copy(z_hbm, zb)
        for i in range(RPT // ZR):
            pltpu.make_async_copy(
                zb, acc.at[pl.ds(sid * RPT + i * ZR, ZR)], ssem).start()
        for i in range(RPT // ZR):
            pltpu.make_async_copy(
                zb, acc.at[pl.ds(sid * RPT + i * ZR, ZR)], ssem).wait()
        pltpu.make_async_copy(src_hbm.at[pl.ds(row0, NG)], src_all,
                              zsem).wait()
        plsc.subcore_barrier()

        def fire(i, carry):
            pltpu.async_copy(ones_v, acc.at[src_all.at[i]], ssem, add=True)
            return carry

        lax.fori_loop(0, NG, fire, 0)

        def drain(i, carry):
            pltpu.make_async_copy(ones_v, acc.at[src_all.at[0]], ssem).wait()
            return carry

        lax.fori_loop(0, NG, drain, 0)
        plsc.subcore_barrier()
        pltpu.sync_copy(acc.at[pl.ds(sid * RPT, RPT)],
                        out_hbm.at[cid, pl.ds(sid * RPT, RPT)])

    return k


@functools.lru_cache(None)
def _prep_call():
    def body(dp_ref, d2_ref):
        deg = dp_ref[0, :, 0:1] + dp_ref[1, :, 0:1]
        d = jnp.where(deg > 0.0, lax.rsqrt(jnp.maximum(deg, 1e-30)), 0.0)
        d2_ref[...] = jnp.broadcast_to(d, (BN, 128))

    return pl.pallas_call(
        body,
        grid=(NP // BN,),
        in_specs=[pl.BlockSpec((2, BN, 16), lambda i: (0, i, 0))],
        out_specs=pl.BlockSpec((BN, 128), lambda i: (i, 0)),
        out_shape=jax.ShapeDtypeStruct((NP, 128), jnp.float32),
    )


@functools.lru_cache(None)
def _y_call(Fin):
    def body(h_ref, b_ref, ws_ref, d2_ref, y0, y1, y2, y3, y4, hws, v_ref):
        h = h_ref[...]
        outs = (y0, y1, y2, y3, y4)
        for j in range(5):
            outs[j][...] = jnp.dot(h, b_ref[j],
                                   preferred_element_type=jnp.float32,
                           precision=lax.Precision.HIGHEST)
        hws[...] = jnp.dot(h, ws_ref[0], preferred_element_type=jnp.float32,
                           precision=lax.Precision.HIGHEST)
        y5 = jnp.dot(h, b_ref[5], preferred_element_type=jnp.float32,
                           precision=lax.Precision.HIGHEST)
        v_ref[...] = d2_ref[...] * y5

    o64 = pl.BlockSpec((BN, 64), lambda i: (i, 0))
    return pl.pallas_call(
        body,
        grid=(NP // BN,),
        in_specs=[
            pl.BlockSpec((BN, Fin), lambda i: (i, 0)),
            pl.BlockSpec((6, Fin, 64), lambda i: (0, 0, 0)),
            pl.BlockSpec((1, Fin, 64), lambda i: (0, 0, 0)),
            o64,
        ],
        out_specs=[o64] * 7,
        out_shape=[jax.ShapeDtypeStruct((NP, 64), jnp.float32)] * 7,
    )


@functools.lru_cache(None)
def _comb_call():
    def body(p_ref, d2_ref, y_ref, v_ref):
        d2 = d2_ref[...]
        z = -d2 * (p_ref[0] + p_ref[1]) + y_ref[...]
        v_ref[...] = d2 * z

    o64 = pl.BlockSpec((BN, 64), lambda i: (i, 0))
    return pl.pallas_call(
        body,
        grid=(NP // BN,),
        in_specs=[pl.BlockSpec((2, BN, 64), lambda i: (0, i, 0)), o64, o64],
        out_specs=o64,
        out_shape=jax.ShapeDtypeStruct((NP, 64), jnp.float32),
    )


@functools.lru_cache(None)
def _combfin_call():
    def body(p_ref, d2_ref, y_ref, hws_ref, bk_ref, bs_ref, h_ref, v_ref):
        d2 = d2_ref[...]
        z = -d2 * (p_ref[0] + p_ref[1]) + y_ref[...]
        h = jnp.maximum(z + bk_ref[...], 0.0) + hws_ref[...] + bs_ref[...]
        h_ref[...] = h
        v_ref[...] = d2 * h

    o64 = pl.BlockSpec((BN, 64), lambda i: (i, 0))
    b64 = pl.BlockSpec((1, 64), lambda i: (0, 0))
    return pl.pallas_call(
        body,
        grid=(NP // BN,),
        in_specs=[pl.BlockSpec((2, BN, 64), lambda i: (0, i, 0)),
                  o64, o64, o64, b64, b64],
        out_specs=[o64, o64],
        out_shape=[jax.ShapeDtypeStruct((NP, 64), jnp.float32)] * 2,
    )


@functools.lru_cache(None)
def _mixprep_call():
    def body(h_ref, x_ref, w1a, w1b, w0a, w0b, bm_ref, d2_ref,
             va_ref, vb_ref, op_ref):
        h = h_ref[...]
        x = x_ref[...]
        u = jnp.dot(h, w1a[...], preferred_element_type=jnp.float32,
                           precision=lax.Precision.HIGHEST)
        u = u + jnp.dot(x, w1b[...], preferred_element_type=jnp.float32,
                           precision=lax.Precision.HIGHEST)
        vu = d2_ref[...] * u
        va_ref[...] = vu[:, :64]
        vb_ref[...] = vu[:, 64:]
        op = jnp.dot(h, w0a[...], preferred_element_type=jnp.float32,
                           precision=lax.Precision.HIGHEST)
        op = op + jnp.dot(x, w0b[...], preferred_element_type=jnp.float32,
                           precision=lax.Precision.HIGHEST)
        op_ref[...] = op + bm_ref[...]

    o64 = pl.BlockSpec((BN, 64), lambda i: (i, 0))
    o128 = pl.BlockSpec((BN, 128), lambda i: (i, 0))
    return pl.pallas_call(
        body,
        grid=(NP // BN,),
        in_specs=[
            o64,
            o128,
            pl.BlockSpec((64, 128), lambda i: (0, 0)),
            pl.BlockSpec((128, 128), lambda i: (0, 0)),
            pl.BlockSpec((64, 128), lambda i: (0, 0)),
            pl.BlockSpec((128, 128), lambda i: (0, 0)),
            pl.BlockSpec((1, 128), lambda i: (0, 0)),
            o128,
        ],
        out_specs=[o64, o64, o128],
        out_shape=[
            jax.ShapeDtypeStruct((NP, 64), jnp.float32),
            jax.ShapeDtypeStruct((NP, 64), jnp.float32),
            jax.ShapeDtypeStruct((NP, 128), jnp.float32),
        ],
    )


@functools.lru_cache(None)
def _mixfin_call():
    def body(pa_ref, pb_ref, d2_ref, op_ref, o_ref):
        s = jnp.concatenate([pa_ref[0] + pa_ref[1], pb_ref[0] + pb_ref[1]],
                            axis=1)
        o_ref[...] = op_ref[...] - d2_ref[...] * s

    o128 = pl.BlockSpec((BN, 128), lambda i: (i, 0))
    p64 = pl.BlockSpec((2, BN, 64), lambda i: (0, i, 0))
    return pl.pallas_call(
        body,
        grid=(NP // BN,),
        in_specs=[p64, p64, o128, o128],
        out_specs=o128,
        out_shape=jax.ShapeDtypeStruct((NP, 128), jnp.float32),
    )


_CHEB_C = [[1, 0, 0, 0, 0, 0],
           [0, 1, 0, 0, 0, 0],
           [-1, 0, 2, 0, 0, 0],
           [0, -3, 0, 4, 0, 0],
           [1, 0, -8, 0, 8, 0],
           [0, 5, 0, -20, 0, 16]]


def kernel(x, edge_index, Wk0, bk0, Ws0, bs0, Wk1, bk1, Ws1, bs1,
           Wk2, bk2, Ws2, bs2, Wm, bm):
    f32 = jnp.float32
    x_pad = jnp.zeros((NP, 128), f32).at[:N].set(x)
    src_pad = jnp.full((EP - E + NBUF * GRP,), NP - 1, jnp.int32)
    dst_pad = jnp.full((EP - E,), NP - 1, jnp.int32)
    src2d = jnp.concatenate([edge_index[0], src_pad]).reshape(-1, GRP)
    dst2d = jnp.concatenate([edge_index[1], dst_pad]).reshape(-1, GRP)
    z64 = jnp.zeros((ZR, 64), f32)
    z16 = jnp.zeros((ZR, 16), f32)
    ones16 = jnp.ones((GRP, 16), f32)
    C = jnp.array(_CHEB_C, f32)

    degparts = _deg_kernel()(src2d, ones16, z16)
    d2 = _prep_call()(degparts)
    d2_64 = d2[:, :64]

    h = x_pad
    blocks = [(Wk0, bk0, Ws0, bs0), (Wk1, bk1, Ws1, bs1), (Wk2, bk2, Ws2, bs2)]
    for bi, (Wk, bk, Ws, bs) in enumerate(blocks):
        Fin = 128 if bi == 0 else 64
        B = jnp.einsum("kj,kfo->jfo", C, Wk)
        y0, y1, y2, y3, y4, hws, v = _y_call(Fin)(h, B, Ws, d2_64)
        ys = (y0, y1, y2, y3, y4)
        for j in range(4, -1, -1):
            parts = _scatter_kernel()(v, src2d, dst2d, z64)
            if j > 0:
                v = _comb_call()(parts, d2_64, ys[j])
            else:
                h, v = _combfin_call()(parts, d2_64, ys[0], hws,
                                       bk.reshape(1, 64), bs.reshape(1, 64))

    vua, vub, out_part = _mixprep_call()(h, x_pad, Wm[1, :64], Wm[1, 64:],
                                         Wm[0, :64], Wm[0, 64:],
                                         bm.reshape(1, 128), d2)
    pua = _scatter_kernel()(vua, src2d, dst2d, z64)
    pub = _scatter_kernel()(vub, src2d, dst2d, z64)
    out = _mixfin_call()(pua, pub, d2, out_part)
    return out[:N]


# speedup vs baseline: 1.0168x; 1.0168x over previous
import functools

import jax
import jax.numpy as jnp
from jax import lax
from jax.experimental import pallas as pl
from jax.experimental.pallas import tpu as pltpu
from jax.experimental.pallas import tpu_sc as plsc

N = 10000
E = 320000
NP = 10240
EP = 331776
NC = 2
NS = 16
GRP = 128
SLAB = 16
SL = (EP // (NC * NS)) // (SLAB * GRP)
RPT = NP // NS
ZR = 64
BN = 512


def _mesh():
    return plsc.VectorSubcoreMesh(core_axis_name="c", subcore_axis_name="s")


NG = EP // (NC * NS) // GRP
NBUF = 3
FS = 64


@functools.lru_cache(None)
def _scatter_kernel(F=FS):
    @functools.partial(
        pl.kernel,
        mesh=_mesh(),
        compiler_params=pltpu.CompilerParams(use_tc_tiling_on_sc=False),
        out_type=jax.ShapeDtypeStruct((NC, NP, F), jnp.float32),
        scratch_types=[
            pltpu.VMEM((NG, GRP), jnp.int32),
            pltpu.VMEM((NG, GRP), jnp.int32),
        ] + [pltpu.VMEM((GRP, F), jnp.float32)] * NBUF + [
            pltpu.VMEM_SHARED((NP, F), jnp.float32),
            pltpu.VMEM_SHARED((NP, F), jnp.float32),
        ] + [pltpu.SemaphoreType.DMA] * (2 * NBUF + 2),
    )
    def k(v_hbm, src_hbm, dst_hbm, z_hbm, out_hbm, src_all, dst_all,
          r0, r1, r2, acc, vtab, g0, g1, g2, s0, s1, s2, zsem, isem):
        rows = (r0, r1, r2)
        gsem = (g0, g1, g2)
        ssem = (s0, s1, s2)
        cid = lax.axis_index("c")
        sid = lax.axis_index("s")
        wid = sid * NC + cid
        row0 = wid * NG
        pltpu.make_async_copy(src_hbm.at[pl.ds(row0, NG)], src_all,
                              isem).start()
        pltpu.make_async_copy(dst_hbm.at[pl.ds(row0, NG)], dst_all,
                              isem).start()
        pltpu.make_async_copy(v_hbm.at[pl.ds(sid * RPT, RPT)],
                              vtab.at[pl.ds(sid * RPT, RPT)], isem).start()
        pltpu.sync_copy(z_hbm, r0.at[pl.ds(0, ZR)])
        for i in range(RPT // ZR):
            pltpu.make_async_copy(
                r0.at[pl.ds(0, ZR)],
                acc.at[pl.ds(sid * RPT + i * ZR, ZR)], zsem).start()
        for i in range(RPT // ZR):
            pltpu.make_async_copy(
                r0.at[pl.ds(0, ZR)],
                acc.at[pl.ds(sid * RPT + i * ZR, ZR)], zsem).wait()
        pltpu.make_async_copy(src_hbm.at[pl.ds(row0, NG)], src_all,
                              isem).wait()
        pltpu.make_async_copy(dst_hbm.at[pl.ds(row0, NG)], dst_all,
                              isem).wait()
        pltpu.make_async_copy(v_hbm.at[pl.ds(sid * RPT, RPT)],
                              vtab.at[pl.ds(sid * RPT, RPT)], isem).wait()
        plsc.subcore_barrier()

        def startg(b, i):
            pltpu.make_async_copy(vtab.at[src_all.at[i]], rows[b],
                                  gsem[b]).start()

        def waitg(b, i):
            pltpu.make_async_copy(vtab.at[src_all.at[i]], rows[b],
                                  gsem[b]).wait()

        def starts(b, i):
            pltpu.make_async_copy(rows[b], acc.at[dst_all.at[i]],
                                  ssem[b]).start(add=True)

        def waits(b, i):
            pltpu.make_async_copy(rows[b], acc.at[dst_all.at[i]],
                                  ssem[b]).wait()

        for b in range(NBUF):
            startg(b, b)
        waitg(0, 0); starts(0, 0)
        waitg(1, 1); starts(1, 1); waits(0, 0); startg(0, 3)

        PAT = ((2, 1), (0, 2), (1, 0))

        def outer(g, carry):
            i0 = 2 + 3 * g
            for u in range(3):
                b, bo = PAT[u]
                i = i0 + u
                waitg(b, i)
                starts(b, i)
                waits(bo, i - 1)
                startg(bo, i + 2)
            return carry

        lax.fori_loop(0, (NG - 6) // 3, outer, 0)
        waitg(2, NG - 4); starts(2, NG - 4); waits(1, NG - 5); startg(1, NG - 2)
        waitg(0, NG - 3); starts(0, NG - 3); waits(2, NG - 4); startg(2, NG - 1)
        waitg(1, NG - 2); starts(1, NG - 2); waits(0, NG - 3)
        waitg(2, NG - 1); starts(2, NG - 1)
        waits(1, NG - 2)
        waits(2, NG - 1)
        plsc.subcore_barrier()
        pltpu.sync_copy(acc.at[pl.ds(sid * RPT, RPT)],
                        out_hbm.at[cid, pl.ds(sid * RPT, RPT)])

    return k


@functools.lru_cache(None)
def _deg_kernel():
    @functools.partial(
        pl.kernel,
        mesh=_mesh(),
        compiler_params=pltpu.CompilerParams(use_tc_tiling_on_sc=False),
        out_type=jax.ShapeDtypeStruct((NC, NP, 16), jnp.float32),
        scratch_types=[
            pltpu.VMEM((NG, GRP), jnp.int32),
            pltpu.VMEM((GRP, 16), jnp.float32),
            pltpu.VMEM((ZR, 16), jnp.float32),
            pltpu.VMEM_SHARED((NP, 16), jnp.float32),
            pltpu.SemaphoreType.DMA,
            pltpu.SemaphoreType.DMA,
        ],
    )
    def k(src_hbm, ones_hbm, z_hbm, out_hbm, src_all, ones_v, zb, acc,
          zsem, ssem):
        cid = lax.axis_index("c")
        sid = lax.axis_index("s")
        wid = sid * NC + cid
        row0 = wid * NG
        pltpu.make_async_copy(src_hbm.at[pl.ds(row0, NG)], src_all,
                              zsem).start()
        pltpu.sync_copy(ones_hbm, ones_v)
        pltpu.sync_copy(z_hbm, zb)
        for i in range(RPT // ZR):
            pltpu.make_async_copy(
                zb, acc.at[pl.ds(sid * RPT + i * ZR, ZR)], ssem).start()
        for i in range(RPT // ZR):
            pltpu.make_async_copy(
                zb, acc.at[pl.ds(sid * RPT + i * ZR, ZR)], ssem).wait()
        pltpu.make_async_copy(src_hbm.at[pl.ds(row0, NG)], src_all,
                              zsem).wait()
        plsc.subcore_barrier()

        def fire(i, carry):
            pltpu.async_copy(ones_v, acc.at[src_all.at[i]], ssem, add=True)
            return carry

        lax.fori_loop(0, NG, fire, 0)

        def drain(i, carry):
            pltpu.make_async_copy(ones_v, acc.at[src_all.at[0]], ssem).wait()
            return carry

        lax.fori_loop(0, NG, drain, 0)
        plsc.subcore_barrier()
        pltpu.sync_copy(acc.at[pl.ds(sid * RPT, RPT)],
                        out_hbm.at[cid, pl.ds(sid * RPT, RPT)])

    return k


@functools.lru_cache(None)
def _prep_call():
    def body(dp_ref, d2_ref):
        deg = dp_ref[0, :, 0:1] + dp_ref[1, :, 0:1]
        d = jnp.where(deg > 0.0, lax.rsqrt(jnp.maximum(deg, 1e-30)), 0.0)
        d2_ref[...] = jnp.broadcast_to(d, (BN, 128))

    return pl.pallas_call(
        body,
        grid=(NP // BN,),
        in_specs=[pl.BlockSpec((2, BN, 16), lambda i: (0, i, 0))],
        out_specs=pl.BlockSpec((BN, 128), lambda i: (i, 0)),
        out_shape=jax.ShapeDtypeStruct((NP, 128), jnp.float32),
    )


@functools.lru_cache(None)
def _y_call(Fin):
    def body(h_ref, b_ref, ws_ref, d2_ref, y0, y1, y2, y3, y4, hws, v_ref):
        h = h_ref[...]
        outs = (y0, y1, y2, y3, y4)
        for j in range(5):
            outs[j][...] = jnp.dot(h, b_ref[j],
                                   preferred_element_type=jnp.float32,
                           precision=lax.Precision.HIGHEST)
        hws[...] = jnp.dot(h, ws_ref[0], preferred_element_type=jnp.float32,
                           precision=lax.Precision.HIGHEST)
        y5 = jnp.dot(h, b_ref[5], preferred_element_type=jnp.float32,
                           precision=lax.Precision.HIGHEST)
        v_ref[...] = d2_ref[...] * y5

    o64 = pl.BlockSpec((BN, 64), lambda i: (i, 0))
    return pl.pallas_call(
        body,
        grid=(NP // BN,),
        in_specs=[
            pl.BlockSpec((BN, Fin), lambda i: (i, 0)),
            pl.BlockSpec((6, Fin, 64), lambda i: (0, 0, 0)),
            pl.BlockSpec((1, Fin, 64), lambda i: (0, 0, 0)),
            o64,
        ],
        out_specs=[o64] * 7,
        out_shape=[jax.ShapeDtypeStruct((NP, 64), jnp.float32)] * 7,
    )


@functools.lru_cache(None)
def _comb_call():
    def body(p_ref, d2_ref, y_ref, v_ref):
        d2 = d2_ref[...]
        z = -d2 * (p_ref[0] + p_ref[1]) + y_ref[...]
        v_ref[...] = d2 * z

    o64 = pl.BlockSpec((BN, 64), lambda i: (i, 0))
    return pl.pallas_call(
        body,
        grid=(NP // BN,),
        in_specs=[pl.BlockSpec((2, BN, 64), lambda i: (0, i, 0)), o64, o64],
        out_specs=o64,
        out_shape=jax.ShapeDtypeStruct((NP, 64), jnp.float32),
    )


@functools.lru_cache(None)
def _combfin_call():
    def body(p_ref, d2_ref, y_ref, hws_ref, bk_ref, bs_ref, h_ref, v_ref):
        d2 = d2_ref[...]
        z = -d2 * (p_ref[0] + p_ref[1]) + y_ref[...]
        h = jnp.maximum(z + bk_ref[...], 0.0) + hws_ref[...] + bs_ref[...]
        h_ref[...] = h
        v_ref[...] = d2 * h

    o64 = pl.BlockSpec((BN, 64), lambda i: (i, 0))
    b64 = pl.BlockSpec((1, 64), lambda i: (0, 0))
    return pl.pallas_call(
        body,
        grid=(NP // BN,),
        in_specs=[pl.BlockSpec((2, BN, 64), lambda i: (0, i, 0)),
                  o64, o64, o64, b64, b64],
        out_specs=[o64, o64],
        out_shape=[jax.ShapeDtypeStruct((NP, 64), jnp.float32)] * 2,
    )


@functools.lru_cache(None)
def _mixprep_call():
    def body(h_ref, x_ref, w1a, w1b, w0a, w0b, bm_ref, d2_ref,
             va_ref, vb_ref, op_ref):
        h = h_ref[...]
        x = x_ref[...]
        u = jnp.dot(h, w1a[...], preferred_element_type=jnp.float32,
                           precision=lax.Precision.HIGHEST)
        u = u + jnp.dot(x, w1b[...], preferred_element_type=jnp.float32,
                           precision=lax.Precision.HIGHEST)
        vu = d2_ref[...] * u
        va_ref[...] = vu[:, :64]
        vb_ref[...] = vu[:, 64:]
        op = jnp.dot(h, w0a[...], preferred_element_type=jnp.float32,
                           precision=lax.Precision.HIGHEST)
        op = op + jnp.dot(x, w0b[...], preferred_element_type=jnp.float32,
                           precision=lax.Precision.HIGHEST)
        op_ref[...] = op + bm_ref[...]

    o64 = pl.BlockSpec((BN, 64), lambda i: (i, 0))
    o128 = pl.BlockSpec((BN, 128), lambda i: (i, 0))
    return pl.pallas_call(
        body,
        grid=(NP // BN,),
        in_specs=[
            o64,
            o128,
            pl.BlockSpec((64, 128), lambda i: (0, 0)),
            pl.BlockSpec((128, 128), lambda i: (0, 0)),
            pl.BlockSpec((64, 128), lambda i: (0, 0)),
            pl.BlockSpec((128, 128), lambda i: (0, 0)),
            pl.BlockSpec((1, 128), lambda i: (0, 0)),
            o128,
        ],
        out_specs=[o64, o64, o128],
        out_shape=[
            jax.ShapeDtypeStruct((NP, 64), jnp.float32),
            jax.ShapeDtypeStruct((NP, 64), jnp.float32),
            jax.ShapeDtypeStruct((NP, 128), jnp.float32),
        ],
    )


@functools.lru_cache(None)
def _mixfin_call():
    def body(pa_ref, pb_ref, d2_ref, op_ref, o_ref):
        s = jnp.concatenate([pa_ref[0] + pa_ref[1], pb_ref[0] + pb_ref[1]],
                            axis=1)
        o_ref[...] = op_ref[...] - d2_ref[...] * s

    o128 = pl.BlockSpec((BN, 128), lambda i: (i, 0))
    p64 = pl.BlockSpec((2, BN, 64), lambda i: (0, i, 0))
    return pl.pallas_call(
        body,
        grid=(NP // BN,),
        in_specs=[p64, p64, o128, o128],
        out_specs=o128,
        out_shape=jax.ShapeDtypeStruct((NP, 128), jnp.float32),
    )


_CHEB_C = [[1, 0, 0, 0, 0, 0],
           [0, 1, 0, 0, 0, 0],
           [-1, 0, 2, 0, 0, 0],
           [0, -3, 0, 4, 0, 0],
           [1, 0, -8, 0, 8, 0],
           [0, 5, 0, -20, 0, 16]]


def kernel(x, edge_index, Wk0, bk0, Ws0, bs0, Wk1, bk1, Ws1, bs1,
           Wk2, bk2, Ws2, bs2, Wm, bm):
    f32 = jnp.float32
    x_pad = jnp.zeros((NP, 128), f32).at[:N].set(x)
    pad_idx = jnp.full((EP - E,), NP - 1, jnp.int32)
    src2d = jnp.concatenate([edge_index[0], pad_idx]).reshape(-1, GRP)
    dst2d = jnp.concatenate([edge_index[1], pad_idx]).reshape(-1, GRP)
    z64 = jnp.zeros((ZR, 64), f32)
    z16 = jnp.zeros((ZR, 16), f32)
    ones16 = jnp.ones((GRP, 16), f32)
    C = jnp.array(_CHEB_C, f32)

    degparts = _deg_kernel()(src2d, ones16, z16)
    d2 = _prep_call()(degparts)
    d2_64 = d2[:, :64]

    h = x_pad
    blocks = [(Wk0, bk0, Ws0, bs0), (Wk1, bk1, Ws1, bs1), (Wk2, bk2, Ws2, bs2)]
    for bi, (Wk, bk, Ws, bs) in enumerate(blocks):
        Fin = 128 if bi == 0 else 64
        B = jnp.einsum("kj,kfo->jfo", C, Wk)
        y0, y1, y2, y3, y4, hws, v = _y_call(Fin)(h, B, Ws, d2_64)
        ys = (y0, y1, y2, y3, y4)
        for j in range(4, -1, -1):
            parts = _scatter_kernel()(v, src2d, dst2d, z64)
            if j > 0:
                v = _comb_call()(parts, d2_64, ys[j])
            else:
                h, v = _combfin_call()(parts, d2_64, ys[0], hws,
                                       bk.reshape(1, 64), bs.reshape(1, 64))

    vua, vub, out_part = _mixprep_call()(h, x_pad, Wm[1, :64], Wm[1, 64:],
                                         Wm[0, :64], Wm[0, 64:],
                                         bm.reshape(1, 128), d2)
    pua = _scatter_kernel()(vua, src2d, dst2d, z64)
    pub = _scatter_kernel()(vub, src2d, dst2d, z64)
    out = _mixfin_call()(pua, pub, d2, out_part)
    return out[:N]


# speedup vs baseline: 1.0808x; 1.0630x over previous
import functools

import jax
import jax.numpy as jnp
from jax import lax
from jax.experimental import pallas as pl
from jax.experimental.pallas import tpu as pltpu
from jax.experimental.pallas import tpu_sc as plsc

N = 10000
E = 320000
NP = 10240
EP = 331776
NC = 2
NS = 16
GRP = 128
SLAB = 16
SL = (EP // (NC * NS)) // (SLAB * GRP)
RPT = NP // NS
ZR = 64
BN = 2048


def _mesh():
    return plsc.VectorSubcoreMesh(core_axis_name="c", subcore_axis_name="s")


NG = EP // (NC * NS) // GRP
NBUF = 3
FS = 64


@functools.lru_cache(None)
def _scatter_kernel(F=FS):
    @functools.partial(
        pl.kernel,
        mesh=_mesh(),
        compiler_params=pltpu.CompilerParams(use_tc_tiling_on_sc=False),
        out_type=jax.ShapeDtypeStruct((NC, NP, F), jnp.float32),
        scratch_types=[
            pltpu.VMEM((NG, GRP), jnp.int32),
            pltpu.VMEM((NG, GRP), jnp.int32),
        ] + [pltpu.VMEM((GRP, F), jnp.float32)] * NBUF + [
            pltpu.VMEM_SHARED((NP, F), jnp.float32),
            pltpu.VMEM_SHARED((NP, F), jnp.float32),
        ] + [pltpu.SemaphoreType.DMA] * (2 * NBUF + 2),
    )
    def k(v_hbm, src_hbm, dst_hbm, z_hbm, out_hbm, src_all, dst_all,
          r0, r1, r2, acc, vtab, g0, g1, g2, s0, s1, s2, zsem, isem):
        rows = (r0, r1, r2)
        gsem = (g0, g1, g2)
        ssem = (s0, s1, s2)
        cid = lax.axis_index("c")
        sid = lax.axis_index("s")
        wid = sid * NC + cid
        row0 = wid * NG
        pltpu.make_async_copy(src_hbm.at[pl.ds(row0, NG)], src_all,
                              isem).start()
        pltpu.make_async_copy(dst_hbm.at[pl.ds(row0, NG)], dst_all,
                              isem).start()
        pltpu.make_async_copy(v_hbm.at[pl.ds(sid * RPT, RPT)],
                              vtab.at[pl.ds(sid * RPT, RPT)], isem).start()
        pltpu.sync_copy(z_hbm, r0.at[pl.ds(0, ZR)])
        for i in range(RPT // ZR):
            pltpu.make_async_copy(
                r0.at[pl.ds(0, ZR)],
                acc.at[pl.ds(sid * RPT + i * ZR, ZR)], zsem).start()
        for i in range(RPT // ZR):
            pltpu.make_async_copy(
                r0.at[pl.ds(0, ZR)],
                acc.at[pl.ds(sid * RPT + i * ZR, ZR)], zsem).wait()
        pltpu.make_async_copy(src_hbm.at[pl.ds(row0, NG)], src_all,
                              isem).wait()
        pltpu.make_async_copy(dst_hbm.at[pl.ds(row0, NG)], dst_all,
                              isem).wait()
        pltpu.make_async_copy(v_hbm.at[pl.ds(sid * RPT, RPT)],
                              vtab.at[pl.ds(sid * RPT, RPT)], isem).wait()
        plsc.subcore_barrier()

        def startg(b, i):
            pltpu.make_async_copy(vtab.at[src_all.at[i]], rows[b],
                                  gsem[b]).start()

        def waitg(b, i):
            pltpu.make_async_copy(vtab.at[src_all.at[i]], rows[b],
                                  gsem[b]).wait()

        def starts(b, i):
            pltpu.make_async_copy(rows[b], acc.at[dst_all.at[i]],
                                  ssem[b]).start(add=True)

        def waits(b, i):
            pltpu.make_async_copy(rows[b], acc.at[dst_all.at[i]],
                                  ssem[b]).wait()

        for b in range(NBUF):
            startg(b, b)
        waitg(0, 0); starts(0, 0)
        waitg(1, 1); starts(1, 1); waits(0, 0); startg(0, 3)

        PAT = ((2, 1), (0, 2), (1, 0))

        def outer(g, carry):
            i0 = 2 + 3 * g
            for u in range(3):
                b, bo = PAT[u]
                i = i0 + u
                waitg(b, i)
                starts(b, i)
                waits(bo, i - 1)
                startg(bo, i + 2)
            return carry

        lax.fori_loop(0, (NG - 6) // 3, outer, 0)
        waitg(2, NG - 4); starts(2, NG - 4); waits(1, NG - 5); startg(1, NG - 2)
        waitg(0, NG - 3); starts(0, NG - 3); waits(2, NG - 4); startg(2, NG - 1)
        waitg(1, NG - 2); starts(1, NG - 2); waits(0, NG - 3)
        waitg(2, NG - 1); starts(2, NG - 1)
        waits(1, NG - 2)
        waits(2, NG - 1)
        plsc.subcore_barrier()
        pltpu.sync_copy(acc.at[pl.ds(sid * RPT, RPT)],
                        out_hbm.at[cid, pl.ds(sid * RPT, RPT)])

    return k


@functools.lru_cache(None)
def _deg_kernel():
    @functools.partial(
        pl.kernel,
        mesh=_mesh(),
        compiler_params=pltpu.CompilerParams(use_tc_tiling_on_sc=False),
        out_type=jax.ShapeDtypeStruct((NC, NP, 16), jnp.float32),
        scratch_types=[
            pltpu.VMEM((NG, GRP), jnp.int32),
            pltpu.VMEM((GRP, 16), jnp.float32),
            pltpu.VMEM((ZR, 16), jnp.float32),
            pltpu.VMEM_SHARED((NP, 16), jnp.float32),
            pltpu.SemaphoreType.DMA,
            pltpu.SemaphoreType.DMA,
        ],
    )
    def k(src_hbm, ones_hbm, z_hbm, out_hbm, src_all, ones_v, zb, acc,
          zsem, ssem):
        cid = lax.axis_index("c")
        sid = lax.axis_index("s")
        wid = sid * NC + cid
        row0 = wid * NG
        pltpu.make_async_copy(src_hbm.at[pl.ds(row0, NG)], src_all,
                              zsem).start()
        pltpu.sync_copy(ones_hbm, ones_v)
        pltpu.sync_copy(z_hbm, zb)
        for i in range(RPT // ZR):
            pltpu.make_async_copy(
                zb, acc.at[pl.ds(sid * RPT + i * ZR, ZR)], ssem).start()
        for i in range(RPT // ZR):
            pltpu.make_async_copy(
                zb, acc.at[pl.ds(sid * RPT + i * ZR, ZR)], ssem).wait()
        pltpu.make_async_copy(src_hbm.at[pl.ds(row0, NG)], src_all,
                              zsem).wait()
        plsc.subcore_barrier()

        def fire(i, carry):
            pltpu.async_copy(ones_v, acc.at[src_all.at[i]], ssem, add=True)
            return carry

        lax.fori_loop(0, NG, fire, 0)

        def drain(i, carry):
            pltpu.make_async_copy(ones_v, acc.at[src_all.at[0]], ssem).wait()
            return carry

        lax.fori_loop(0, NG, drain, 0)
        plsc.subcore_barrier()
        pltpu.sync_copy(acc.at[pl.ds(sid * RPT, RPT)],
                        out_hbm.at[cid, pl.ds(sid * RPT, RPT)])

    return k


@functools.lru_cache(None)
def _prep_call():
    def body(dp_ref, d2_ref):
        deg = dp_ref[0, :, 0:1] + dp_ref[1, :, 0:1]
        d = jnp.where(deg > 0.0, lax.rsqrt(jnp.maximum(deg, 1e-30)), 0.0)
        d2_ref[...] = jnp.broadcast_to(d, (BN, 128))

    return pl.pallas_call(
        body,
        grid=(NP // BN,),
        in_specs=[pl.BlockSpec((2, BN, 16), lambda i: (0, i, 0))],
        out_specs=pl.BlockSpec((BN, 128), lambda i: (i, 0)),
        out_shape=jax.ShapeDtypeStruct((NP, 128), jnp.float32),
    )


@functools.lru_cache(None)
def _y_call(Fin):
    def body(h_ref, b_ref, ws_ref, d2_ref, y0, y1, y2, y3, y4, hws, v_ref):
        h = h_ref[...]
        outs = (y0, y1, y2, y3, y4)
        for j in range(5):
            outs[j][...] = jnp.dot(h, b_ref[j],
                                   preferred_element_type=jnp.float32,
                           precision=lax.Precision.HIGHEST)
        hws[...] = jnp.dot(h, ws_ref[0], preferred_element_type=jnp.float32,
                           precision=lax.Precision.HIGHEST)
        y5 = jnp.dot(h, b_ref[5], preferred_element_type=jnp.float32,
                           precision=lax.Precision.HIGHEST)
        v_ref[...] = d2_ref[...] * y5

    o64 = pl.BlockSpec((BN, 64), lambda i: (i, 0))
    return pl.pallas_call(
        body,
        grid=(NP // BN,),
        in_specs=[
            pl.BlockSpec((BN, Fin), lambda i: (i, 0)),
            pl.BlockSpec((6, Fin, 64), lambda i: (0, 0, 0)),
            pl.BlockSpec((1, Fin, 64), lambda i: (0, 0, 0)),
            o64,
        ],
        out_specs=[o64] * 7,
        out_shape=[jax.ShapeDtypeStruct((NP, 64), jnp.float32)] * 7,
    )


@functools.lru_cache(None)
def _comb_call():
    def body(p_ref, d2_ref, y_ref, v_ref):
        d2 = d2_ref[...]
        z = -d2 * (p_ref[0] + p_ref[1]) + y_ref[...]
        v_ref[...] = d2 * z

    o64 = pl.BlockSpec((BN, 64), lambda i: (i, 0))
    return pl.pallas_call(
        body,
        grid=(NP // BN,),
        in_specs=[pl.BlockSpec((2, BN, 64), lambda i: (0, i, 0)), o64, o64],
        out_specs=o64,
        out_shape=jax.ShapeDtypeStruct((NP, 64), jnp.float32),
    )


@functools.lru_cache(None)
def _combfin_call():
    def body(p_ref, d2_ref, y_ref, hws_ref, bk_ref, bs_ref, h_ref, v_ref):
        d2 = d2_ref[...]
        z = -d2 * (p_ref[0] + p_ref[1]) + y_ref[...]
        h = jnp.maximum(z + bk_ref[...], 0.0) + hws_ref[...] + bs_ref[...]
        h_ref[...] = h
        v_ref[...] = d2 * h

    o64 = pl.BlockSpec((BN, 64), lambda i: (i, 0))
    b64 = pl.BlockSpec((1, 64), lambda i: (0, 0))
    return pl.pallas_call(
        body,
        grid=(NP // BN,),
        in_specs=[pl.BlockSpec((2, BN, 64), lambda i: (0, i, 0)),
                  o64, o64, o64, b64, b64],
        out_specs=[o64, o64],
        out_shape=[jax.ShapeDtypeStruct((NP, 64), jnp.float32)] * 2,
    )


@functools.lru_cache(None)
def _mixprep_call():
    def body(h_ref, x_ref, w1a, w1b, w0a, w0b, bm_ref, d2_ref,
             va_ref, vb_ref, op_ref):
        h = h_ref[...]
        x = x_ref[...]
        u = jnp.dot(h, w1a[...], preferred_element_type=jnp.float32,
                           precision=lax.Precision.HIGHEST)
        u = u + jnp.dot(x, w1b[...], preferred_element_type=jnp.float32,
                           precision=lax.Precision.HIGHEST)
        vu = d2_ref[...] * u
        va_ref[...] = vu[:, :64]
        vb_ref[...] = vu[:, 64:]
        op = jnp.dot(h, w0a[...], preferred_element_type=jnp.float32,
                           precision=lax.Precision.HIGHEST)
        op = op + jnp.dot(x, w0b[...], preferred_element_type=jnp.float32,
                           precision=lax.Precision.HIGHEST)
        op_ref[...] = op + bm_ref[...]

    o64 = pl.BlockSpec((BN, 64), lambda i: (i, 0))
    o128 = pl.BlockSpec((BN, 128), lambda i: (i, 0))
    return pl.pallas_call(
        body,
        grid=(NP // BN,),
        in_specs=[
            o64,
            o128,
            pl.BlockSpec((64, 128), lambda i: (0, 0)),
            pl.BlockSpec((128, 128), lambda i: (0, 0)),
            pl.BlockSpec((64, 128), lambda i: (0, 0)),
            pl.BlockSpec((128, 128), lambda i: (0, 0)),
            pl.BlockSpec((1, 128), lambda i: (0, 0)),
            o128,
        ],
        out_specs=[o64, o64, o128],
        out_shape=[
            jax.ShapeDtypeStruct((NP, 64), jnp.float32),
            jax.ShapeDtypeStruct((NP, 64), jnp.float32),
            jax.ShapeDtypeStruct((NP, 128), jnp.float32),
        ],
    )


@functools.lru_cache(None)
def _mixfin_call():
    def body(pa_ref, pb_ref, d2_ref, op_ref, o_ref):
        s = jnp.concatenate([pa_ref[0] + pa_ref[1], pb_ref[0] + pb_ref[1]],
                            axis=1)
        o_ref[...] = op_ref[...] - d2_ref[...] * s

    o128 = pl.BlockSpec((BN, 128), lambda i: (i, 0))
    p64 = pl.BlockSpec((2, BN, 64), lambda i: (0, i, 0))
    return pl.pallas_call(
        body,
        grid=(NP // BN,),
        in_specs=[p64, p64, o128, o128],
        out_specs=o128,
        out_shape=jax.ShapeDtypeStruct((NP, 128), jnp.float32),
    )


_CHEB_C = [[1, 0, 0, 0, 0, 0],
           [0, 1, 0, 0, 0, 0],
           [-1, 0, 2, 0, 0, 0],
           [0, -3, 0, 4, 0, 0],
           [1, 0, -8, 0, 8, 0],
           [0, 5, 0, -20, 0, 16]]


def kernel(x, edge_index, Wk0, bk0, Ws0, bs0, Wk1, bk1, Ws1, bs1,
           Wk2, bk2, Ws2, bs2, Wm, bm):
    f32 = jnp.float32
    x_pad = jnp.zeros((NP, 128), f32).at[:N].set(x)
    pad_idx = jnp.full((EP - E,), NP - 1, jnp.int32)
    src2d = jnp.concatenate([edge_index[0], pad_idx]).reshape(-1, GRP)
    dst2d = jnp.concatenate([edge_index[1], pad_idx]).reshape(-1, GRP)
    z64 = jnp.zeros((ZR, 64), f32)
    z16 = jnp.zeros((ZR, 16), f32)
    ones16 = jnp.ones((GRP, 16), f32)
    C = jnp.array(_CHEB_C, f32)

    degparts = _deg_kernel()(src2d, ones16, z16)
    d2 = _prep_call()(degparts)
    d2_64 = d2[:, :64]

    h = x_pad
    blocks = [(Wk0, bk0, Ws0, bs0), (Wk1, bk1, Ws1, bs1), (Wk2, bk2, Ws2, bs2)]
    for bi, (Wk, bk, Ws, bs) in enumerate(blocks):
        Fin = 128 if bi == 0 else 64
        B = jnp.einsum("kj,kfo->jfo", C, Wk)
        y0, y1, y2, y3, y4, hws, v = _y_call(Fin)(h, B, Ws, d2_64)
        ys = (y0, y1, y2, y3, y4)
        for j in range(4, -1, -1):
            parts = _scatter_kernel()(v, src2d, dst2d, z64)
            if j > 0:
                v = _comb_call()(parts, d2_64, ys[j])
            else:
                h, v = _combfin_call()(parts, d2_64, ys[0], hws,
                                       bk.reshape(1, 64), bs.reshape(1, 64))

    vua, vub, out_part = _mixprep_call()(h, x_pad, Wm[1, :64], Wm[1, 64:],
                                         Wm[0, :64], Wm[0, 64:],
                                         bm.reshape(1, 128), d2)
    pua = _scatter_kernel()(vua, src2d, dst2d, z64)
    pub = _scatter_kernel()(vub, src2d, dst2d, z64)
    out = _mixfin_call()(pua, pub, d2, out_part)
    return out[:N]
